# Initial kernel scaffold; baseline (speedup 1.0000x reference)
import jax
import jax.numpy as jnp
from jax.experimental import pallas as pl


def kernel(f_start, f_end, class_logits, confidence, gt_boxes):
    raise NotImplementedError("write your pallas kernel here")


# speedup vs baseline: 2.6461x; 2.6461x over previous
import functools

import jax
import jax.numpy as jnp
from jax import lax
from jax.experimental import pallas as pl
from jax.experimental.pallas import tpu as pltpu
from jax.experimental.pallas import tpu_sc as plsc

_VL = 1000000.0
_EPS = 1e-08
_NT = 10
_LANES = 16
_NC = 2
_NS = 16


def _make_sc_candidates(B, N, CHUNK):
    VECS = CHUNK // _LANES
    mesh = plsc.VectorSubcoreMesh(core_axis_name="c", subcore_axis_name="s")

    @functools.partial(
        pl.kernel,
        out_type=[
            jax.ShapeDtypeStruct((B * 2, 6, _NT, _LANES), jnp.float32),
            jax.ShapeDtypeStruct((B * 2, _NT, _LANES), jnp.int32),
        ],
        mesh=mesh,
        compiler_params=pltpu.CompilerParams(
            use_tc_tiling_on_sc=False, needs_layout_passes=False),
        scratch_types=[
            pltpu.VMEM((CHUNK,), jnp.float32),
            pltpu.VMEM((CHUNK,), jnp.float32),
            pltpu.VMEM((CHUNK,), jnp.float32),
            pltpu.VMEM((CHUNK,), jnp.float32),
            pltpu.VMEM((CHUNK,), jnp.float32),
            pltpu.VMEM((CHUNK,), jnp.float32),
            pltpu.VMEM((CHUNK,), jnp.float32),
            pltpu.VMEM((CHUNK,), jnp.float32),
            pltpu.VMEM((CHUNK,), jnp.float32),
            pltpu.VMEM((_LANES,), jnp.float32),
            pltpu.VMEM((_LANES,), jnp.float32),
            pltpu.VMEM((_LANES,), jnp.float32),
            pltpu.VMEM((6, _NT, _LANES), jnp.float32),
            pltpu.VMEM((_NT, _LANES), jnp.int32),
            pltpu.SMEM((1,), jnp.float32),
        ],
    )
    def sck(fs_h, fe_h, c0_h, c1_h, cf_h, gs_h, ge_h, gc_h, outf_h, outi_h,
            fs_v, fe_v, c0_v, c1_v, cf_v, ss_v, se_v, s0_v, s1_v,
            gs_v, ge_v, gc_v, stf_v, sti_v, thr_s):
        b = lax.axis_index("s")
        half = lax.axis_index("c")
        wid = b * 2 + half
        start = half * CHUNK

        pltpu.sync_copy(fs_h.at[b, pl.ds(start, CHUNK)], fs_v)
        pltpu.sync_copy(fe_h.at[b, pl.ds(start, CHUNK)], fe_v)
        pltpu.sync_copy(c0_h.at[b, pl.ds(start, CHUNK)], c0_v)
        pltpu.sync_copy(c1_h.at[b, pl.ds(start, CHUNK)], c1_v)
        pltpu.sync_copy(cf_h.at[b, pl.ds(start, CHUNK)], cf_v)
        pltpu.sync_copy(gs_h.at[b], gs_v)
        pltpu.sync_copy(ge_h.at[b], ge_v)
        pltpu.sync_copy(gc_h.at[b], gc_v)

        def sig_body(i, c):
            sl = pl.ds(i * _LANES, _LANES)
            ss_v[sl] = 1.0 / (1.0 + jnp.exp(-fs_v[sl]))
            se_v[sl] = 1.0 / (1.0 + jnp.exp(-fe_v[sl]))
            s0_v[sl] = 1.0 / (1.0 + jnp.exp(-c0_v[sl]))
            s1_v[sl] = 1.0 / (1.0 + jnp.exp(-c1_v[sl]))
            return c

        lax.fori_loop(0, VECS, sig_body, 0)

        lane_iota = lax.iota(jnp.int32, _LANES)
        gs_all = gs_v[...]
        ge_all = ge_v[...]
        gc_all = gc_v[...]

        for t in range(_NT):
            gsr = gs_all[t]
            ger = ge_all[t]
            gcr = gc_all[t]
            pres = (gsr == gsr) & (ger == ger) & (gcr == gcr)
            gsn = jnp.where(gsr == gsr, gsr, 0.0)
            gen = jnp.where(ger == ger, ger, 0.0)
            gcn = jnp.where(gcr == gcr, gcr, 0.0)
            o0 = (gcn < 1.0).astype(jnp.float32)
            o1 = 1.0 - o0

            stf_v[5, t] = jnp.full((_LANES,), _VL, jnp.float32)
            sti_v[t] = jnp.zeros((_LANES,), jnp.int32) + start
            thr_s[0] = jnp.float32(_VL)

            @pl.when(pres)
            def _row():
                def step(i, c):
                    sl = pl.ds(i * _LANES, _LANES)
                    d0 = ss_v[sl] - gsn
                    d1 = se_v[sl] - gen
                    d2 = s0_v[sl] - o0
                    d3 = s1_v[sl] - o1
                    cost = (d0 * d0 + d1 * d1) + (d2 * d2 + d3 * d3)
                    idx = lane_iota + (start + i * _LANES)
                    mn = jnp.min(cost)

                    @pl.when(mn < thr_s[0])
                    def _merge():
                        bv0 = stf_v[5, t]
                        bi0 = sti_v[t]
                        sv, si = plsc.sort_key_val(cost, idx)
                        rv = lax.rev(sv, (0,))
                        ri = lax.rev(si, (0,))
                        keep = bv0 <= rv
                        mv = jnp.where(keep, bv0, rv)
                        mi = jnp.where(keep, bi0, ri)
                        nbv, nbi = plsc.sort_key_val(mv, mi)
                        stf_v[5, t] = nbv
                        sti_v[t] = nbi
                        thr_s[0] = jnp.max(nbv)

                    return c

                lax.fori_loop(0, VECS, step, 0)

            bi = sti_v[t]
            li = bi - start
            stf_v[0, t] = plsc.load_gather(fs_v, [li])
            stf_v[1, t] = plsc.load_gather(fe_v, [li])
            stf_v[2, t] = plsc.load_gather(c0_v, [li])
            stf_v[3, t] = plsc.load_gather(c1_v, [li])
            stf_v[4, t] = plsc.load_gather(cf_v, [li])
        pltpu.sync_copy(stf_v, outf_h.at[wid])
        pltpu.sync_copy(sti_v, outi_h.at[wid])

    return sck


def _tc_body(N, B, K2, cf_ref, ci_ref, gs_ref, ge_ref, gc_ref, out_ref):
    fs = cf_ref[0]
    fe = cf_ref[1]
    x0 = cf_ref[2]
    x1 = cf_ref[3]
    cfd = cf_ref[4]
    col = ci_ref[...]
    gs_raw = gs_ref[...]
    ge_raw = ge_ref[...]
    gc_raw = gc_ref[...]

    pres = (gs_raw == gs_raw) & (ge_raw == ge_raw) & (gc_raw == gc_raw)
    gs = jnp.where(gs_raw == gs_raw, gs_raw, 0.0)
    ge = jnp.where(ge_raw == ge_raw, ge_raw, 0.0)
    gc = jnp.where(gc_raw == gc_raw, gc_raw, 0.0)
    o0 = (gc < 1.0).astype(jnp.float32)
    o1 = (gc >= 1.0).astype(jnp.float32)

    ss = 1.0 / (1.0 + jnp.exp(-fs))
    se = 1.0 / (1.0 + jnp.exp(-fe))
    s0 = 1.0 / (1.0 + jnp.exp(-x0))
    s1 = 1.0 / (1.0 + jnp.exp(-x1))

    cost = ((ss - gs) ** 2 + (se - ge) ** 2) + (
        (s0 - o0) ** 2 + (s1 - o1) ** 2)
    cost = jnp.where(pres, cost, _VL)

    t_io = lax.broadcasted_iota(jnp.int32, (B, _NT, K2), 1)
    rank = t_io * N + col
    BIG = jnp.int32(2 ** 30)
    r_io = lax.broadcasted_iota(jnp.int32, (B, _NT, 1), 1)

    def rnd(_, carry):
        cost, accl, accc, accf, accw = carry
        mn = jnp.min(cost, axis=(1, 2), keepdims=True)
        key = jnp.where(cost == mn, rank, BIG)
        pick = jnp.min(key, axis=(1, 2), keepdims=True)
        w = (mn < _VL * 0.5).astype(jnp.float32)
        r = pick // N
        c = pick - r * N
        selF = (key == pick).astype(jnp.float32)

        ps = jnp.sum(ss * selF, axis=(1, 2), keepdims=True)
        pe = jnp.sum(se * selF, axis=(1, 2), keepdims=True)
        xa = jnp.sum(x0 * selF, axis=(1, 2), keepdims=True)
        xb = jnp.sum(x1 * selF, axis=(1, 2), keepdims=True)
        cs = jnp.sum(cfd * selF, axis=(1, 2), keepdims=True)

        rsel = (r_io == r).astype(jnp.float32)
        ts = jnp.sum(gs * rsel, axis=(1, 2), keepdims=True)
        te = jnp.sum(ge * rsel, axis=(1, 2), keepdims=True)
        z0 = jnp.sum(o0 * rsel, axis=(1, 2), keepdims=True)
        z1 = jnp.sum(o1 * rsel, axis=(1, 2), keepdims=True)

        accl = accl + w * ((ps - ts) ** 2 + (pe - te) ** 2)

        bce0 = jnp.maximum(xa, 0.0) - xa * z0 + jnp.log1p(jnp.exp(-jnp.abs(xa)))
        bce1 = jnp.maximum(xb, 0.0) - xb * z1 + jnp.log1p(jnp.exp(-jnp.abs(xb)))
        accc = accc + w * (bce0 + bce1)

        a1 = jnp.minimum(ps, pe)
        b1 = jnp.maximum(ps, pe)
        a2 = jnp.minimum(ts, te)
        b2 = jnp.maximum(ts, te)
        inter = jnp.maximum(0.0, jnp.minimum(b1, b2) - jnp.maximum(a1, a2))
        union = jnp.maximum(_EPS, jnp.maximum(b1, b2) - jnp.minimum(a1, a2))
        iou = inter / union
        cp = 1.0 / (1.0 + jnp.exp(-cs))
        accf = accf + w * (cp - iou) ** 2
        accw = accw + w

        cost = jnp.where((t_io == r) | (col == c), _VL, cost)
        return cost, accl, accc, accf, accw

    z = jnp.zeros((B, 1, 1), jnp.float32)
    _, accl, accc, accf, accw = lax.fori_loop(0, _NT, rnd, (cost, z, z, z, z))

    tl = jnp.sum(accl)
    tc = jnp.sum(accc)
    tf = jnp.sum(accf)
    tm = jnp.sum(accw)
    denom = tm + jnp.float32(_EPS)
    loss = tl / denom + tc / denom + tf / denom
    out_ref[...] = jnp.reshape(jnp.where(tm > 0, loss, jnp.float32(0.0)), (1, 1))


def kernel(f_start, f_end, class_logits, confidence, gt_boxes):
    B, N = f_start.shape
    CHUNK = N // 2

    cl0 = class_logits[..., 0]
    cl1 = class_logits[..., 1]
    gsb = gt_boxes[..., 0]
    geb = gt_boxes[..., 1]
    gcb = gt_boxes[..., 2]
    padn = _LANES - gsb.shape[1]
    pad = jnp.zeros((B, padn), jnp.float32)
    gs16 = jnp.concatenate([gsb, pad], axis=1)
    ge16 = jnp.concatenate([geb, pad], axis=1)
    gc16 = jnp.concatenate([gcb, pad], axis=1)

    sck = _make_sc_candidates(B, N, CHUNK)
    outf, outi = sck(f_start, f_end, cl0, cl1, confidence, gs16, ge16, gc16)

    K2 = 2 * _LANES
    cf = outf.reshape(B, 2, 6, _NT, _LANES).transpose(2, 0, 3, 1, 4).reshape(
        6, B, _NT, K2)
    ci = outi.reshape(B, 2, _NT, _LANES).transpose(0, 2, 1, 3).reshape(
        B, _NT, K2)

    loss2d = pl.pallas_call(
        functools.partial(_tc_body, N, B, K2),
        out_shape=jax.ShapeDtypeStruct((1, 1), jnp.float32),
    )(cf, ci, gsb[..., None], geb[..., None], gcb[..., None])
    return loss2d[0, 0]


# speedup vs baseline: 7.5489x; 2.8528x over previous
import functools

import jax
import jax.numpy as jnp
from jax import lax
from jax.experimental import pallas as pl
from jax.experimental.pallas import tpu as pltpu
from jax.experimental.pallas import tpu_sc as plsc

_VL = 1000000.0
_EPS = 1e-08
_NT = 10
_LANES = 16
_NC = 2
_NS = 16


def _make_sc_candidates(B, N, CHUNK):
    VECS = CHUNK // _LANES
    mesh = plsc.VectorSubcoreMesh(core_axis_name="c", subcore_axis_name="s")

    @functools.partial(
        pl.kernel,
        out_type=[
            jax.ShapeDtypeStruct((B * 2, 6, _NT, _LANES), jnp.float32),
            jax.ShapeDtypeStruct((B * 2, _NT, _LANES), jnp.int32),
        ],
        mesh=mesh,
        compiler_params=pltpu.CompilerParams(
            use_tc_tiling_on_sc=False, needs_layout_passes=False),
        scratch_types=[
            pltpu.VMEM((CHUNK,), jnp.float32),
            pltpu.VMEM((CHUNK,), jnp.float32),
            pltpu.VMEM((CHUNK,), jnp.float32),
            pltpu.VMEM((CHUNK,), jnp.float32),
            pltpu.VMEM((CHUNK,), jnp.float32),
            pltpu.VMEM((CHUNK,), jnp.float32),
            pltpu.VMEM((CHUNK,), jnp.float32),
            pltpu.VMEM((CHUNK,), jnp.float32),
            pltpu.VMEM((CHUNK,), jnp.float32),
            pltpu.VMEM((_LANES,), jnp.float32),
            pltpu.VMEM((_LANES,), jnp.float32),
            pltpu.VMEM((_LANES,), jnp.float32),
            pltpu.VMEM((6, _NT, _LANES), jnp.float32),
            pltpu.VMEM((_NT, _LANES), jnp.int32),
            pltpu.SMEM((_NT,), jnp.float32),
        ],
    )
    def sck(fs_h, fe_h, c0_h, c1_h, cf_h, gs_h, ge_h, gc_h, outf_h, outi_h,
            fs_v, fe_v, c0_v, c1_v, cf_v, ss_v, se_v, d_v, qq_v,
            gs_v, ge_v, gc_v, stf_v, sti_v, thr_s):
        b = lax.axis_index("s")
        half = lax.axis_index("c")
        wid = b * 2 + half
        start = half * CHUNK

        pltpu.sync_copy(fs_h.at[b, pl.ds(start, CHUNK)], fs_v)
        pltpu.sync_copy(fe_h.at[b, pl.ds(start, CHUNK)], fe_v)
        pltpu.sync_copy(c0_h.at[b, pl.ds(start, CHUNK)], c0_v)
        pltpu.sync_copy(c1_h.at[b, pl.ds(start, CHUNK)], c1_v)
        pltpu.sync_copy(cf_h.at[b, pl.ds(start, CHUNK)], cf_v)
        pltpu.sync_copy(gs_h.at[b], gs_v)
        pltpu.sync_copy(ge_h.at[b], ge_v)
        pltpu.sync_copy(gc_h.at[b], gc_v)

        def sig_body(i, c):
            sl = pl.ds(i * _LANES, _LANES)
            es = 1.0 / (1.0 + jnp.exp(-fs_v[sl]))
            ee = 1.0 / (1.0 + jnp.exp(-fe_v[sl]))
            e0 = 1.0 / (1.0 + jnp.exp(-c0_v[sl]))
            e1 = 1.0 / (1.0 + jnp.exp(-c1_v[sl]))
            ss_v[sl] = es
            se_v[sl] = ee
            d_v[sl] = e0 - e1
            qq_v[sl] = (es * es + ee * ee) + (e0 * e0 + e1 * (e1 - 2.0))
            return c

        lax.fori_loop(0, VECS, sig_body, 0)

        lane_iota = lax.iota(jnp.int32, _LANES)
        gs_all = gs_v[...]
        ge_all = ge_v[...]
        gc_all = gc_v[...]

        rowa = []
        rowb = []
        rowc = []
        for t in range(_NT):
            gsr = gs_all[t]
            ger = ge_all[t]
            gcr = gc_all[t]
            pres = (gsr == gsr) & (ger == ger) & (gcr == gcr)
            gsn = jnp.where(gsr == gsr, gsr, 0.0)
            gen = jnp.where(ger == ger, ger, 0.0)
            gcn = jnp.where(gcr == gcr, gcr, 0.0)
            o0 = (gcn < 1.0).astype(jnp.float32)
            rowa.append(-2.0 * gsn)
            rowb.append(-2.0 * gen)
            rowc.append(-2.0 * o0)
            stf_v[5, t] = jnp.full((_LANES,), _VL, jnp.float32)
            sti_v[t] = jnp.zeros((_LANES,), jnp.int32) + start
            thr_s[t] = jnp.where(pres, jnp.float32(_VL), jnp.float32(-_VL))

        def step(i, c):
            sl = pl.ds(i * _LANES, _LANES)
            ss = ss_v[sl]
            se = se_v[sl]
            dd = d_v[sl]
            qq = qq_v[sl]
            costs = []
            preds = []
            for t in range(_NT):
                cost_t = qq + (rowa[t] * ss + rowb[t] * se) + rowc[t] * dd
                costs.append(cost_t)
                preds.append(jnp.any(cost_t < thr_s[t]))
            anyp = preds[0]
            for t in range(1, _NT):
                anyp = anyp | preds[t]

            @pl.when(anyp)
            def _any_merge():
                idx = lane_iota + (start + i * _LANES)
                for t in range(_NT):
                    @pl.when(preds[t])
                    def _merge(t=t, cost=costs[t]):
                        bv0 = stf_v[5, t]
                        bi0 = sti_v[t]
                        sv, si = plsc.sort_key_val(cost, idx)
                        rv = lax.rev(sv, (0,))
                        ri = lax.rev(si, (0,))
                        keep = bv0 <= rv
                        mv = jnp.where(keep, bv0, rv)
                        mi = jnp.where(keep, bi0, ri)
                        nbv, nbi = plsc.sort_key_val(mv, mi)
                        stf_v[5, t] = nbv
                        sti_v[t] = nbi
                        thr_s[t] = jnp.max(nbv)

            return c

        lax.fori_loop(0, VECS, step, 0)

        for t in range(_NT):
            bi = sti_v[t]
            li = bi - start
            stf_v[0, t] = plsc.load_gather(fs_v, [li])
            stf_v[1, t] = plsc.load_gather(fe_v, [li])
            stf_v[2, t] = plsc.load_gather(c0_v, [li])
            stf_v[3, t] = plsc.load_gather(c1_v, [li])
            stf_v[4, t] = plsc.load_gather(cf_v, [li])
        pltpu.sync_copy(stf_v, outf_h.at[wid])
        pltpu.sync_copy(sti_v, outi_h.at[wid])

    return sck


def _tc_body(N, B, K2, cf_ref, ci_ref, gs_ref, ge_ref, gc_ref, out_ref):
    fs = cf_ref[0]
    fe = cf_ref[1]
    x0 = cf_ref[2]
    x1 = cf_ref[3]
    cfd = cf_ref[4]
    col = ci_ref[...]
    gs_raw = gs_ref[...]
    ge_raw = ge_ref[...]
    gc_raw = gc_ref[...]

    pres = (gs_raw == gs_raw) & (ge_raw == ge_raw) & (gc_raw == gc_raw)
    gs = jnp.where(gs_raw == gs_raw, gs_raw, 0.0)
    ge = jnp.where(ge_raw == ge_raw, ge_raw, 0.0)
    gc = jnp.where(gc_raw == gc_raw, gc_raw, 0.0)
    o0 = (gc < 1.0).astype(jnp.float32)
    o1 = (gc >= 1.0).astype(jnp.float32)

    ss = 1.0 / (1.0 + jnp.exp(-fs))
    se = 1.0 / (1.0 + jnp.exp(-fe))
    s0 = 1.0 / (1.0 + jnp.exp(-x0))
    s1 = 1.0 / (1.0 + jnp.exp(-x1))

    cost = ((ss - gs) ** 2 + (se - ge) ** 2) + (
        (s0 - o0) ** 2 + (s1 - o1) ** 2)
    cost = jnp.where(pres, cost, _VL)

    t_io = lax.broadcasted_iota(jnp.int32, (B, _NT, K2), 1)
    rank = t_io * N + col
    BIG = jnp.int32(2 ** 30)
    r_io = lax.broadcasted_iota(jnp.int32, (B, _NT, 1), 1)

    def rnd(_, carry):
        cost, accl, accc, accf, accw = carry
        mn = jnp.min(cost, axis=(1, 2), keepdims=True)
        key = jnp.where(cost == mn, rank, BIG)
        pick = jnp.min(key, axis=(1, 2), keepdims=True)
        w = (mn < _VL * 0.5).astype(jnp.float32)
        r = pick // N
        c = pick - r * N
        selF = (key == pick).astype(jnp.float32)

        ps = jnp.sum(ss * selF, axis=(1, 2), keepdims=True)
        pe = jnp.sum(se * selF, axis=(1, 2), keepdims=True)
        xa = jnp.sum(x0 * selF, axis=(1, 2), keepdims=True)
        xb = jnp.sum(x1 * selF, axis=(1, 2), keepdims=True)
        cs = jnp.sum(cfd * selF, axis=(1, 2), keepdims=True)

        rsel = (r_io == r).astype(jnp.float32)
        ts = jnp.sum(gs * rsel, axis=(1, 2), keepdims=True)
        te = jnp.sum(ge * rsel, axis=(1, 2), keepdims=True)
        z0 = jnp.sum(o0 * rsel, axis=(1, 2), keepdims=True)
        z1 = jnp.sum(o1 * rsel, axis=(1, 2), keepdims=True)

        accl = accl + w * ((ps - ts) ** 2 + (pe - te) ** 2)

        bce0 = jnp.maximum(xa, 0.0) - xa * z0 + jnp.log1p(jnp.exp(-jnp.abs(xa)))
        bce1 = jnp.maximum(xb, 0.0) - xb * z1 + jnp.log1p(jnp.exp(-jnp.abs(xb)))
        accc = accc + w * (bce0 + bce1)

        a1 = jnp.minimum(ps, pe)
        b1 = jnp.maximum(ps, pe)
        a2 = jnp.minimum(ts, te)
        b2 = jnp.maximum(ts, te)
        inter = jnp.maximum(0.0, jnp.minimum(b1, b2) - jnp.maximum(a1, a2))
        union = jnp.maximum(_EPS, jnp.maximum(b1, b2) - jnp.minimum(a1, a2))
        iou = inter / union
        cp = 1.0 / (1.0 + jnp.exp(-cs))
        accf = accf + w * (cp - iou) ** 2
        accw = accw + w

        cost = jnp.where((t_io == r) | (col == c), _VL, cost)
        return cost, accl, accc, accf, accw

    z = jnp.zeros((B, 1, 1), jnp.float32)
    _, accl, accc, accf, accw = lax.fori_loop(0, _NT, rnd, (cost, z, z, z, z))

    tl = jnp.sum(accl)
    tc = jnp.sum(accc)
    tf = jnp.sum(accf)
    tm = jnp.sum(accw)
    denom = tm + jnp.float32(_EPS)
    loss = tl / denom + tc / denom + tf / denom
    out_ref[...] = jnp.reshape(jnp.where(tm > 0, loss, jnp.float32(0.0)), (1, 1))


def kernel(f_start, f_end, class_logits, confidence, gt_boxes):
    B, N = f_start.shape
    CHUNK = N // 2

    cl0 = class_logits[..., 0]
    cl1 = class_logits[..., 1]
    gsb = gt_boxes[..., 0]
    geb = gt_boxes[..., 1]
    gcb = gt_boxes[..., 2]
    padn = _LANES - gsb.shape[1]
    pad = jnp.zeros((B, padn), jnp.float32)
    gs16 = jnp.concatenate([gsb, pad], axis=1)
    ge16 = jnp.concatenate([geb, pad], axis=1)
    gc16 = jnp.concatenate([gcb, pad], axis=1)

    sck = _make_sc_candidates(B, N, CHUNK)
    outf, outi = sck(f_start, f_end, cl0, cl1, confidence, gs16, ge16, gc16)

    K2 = 2 * _LANES
    cf = outf.reshape(B, 2, 6, _NT, _LANES).transpose(2, 0, 3, 1, 4).reshape(
        6, B, _NT, K2)
    ci = outi.reshape(B, 2, _NT, _LANES).transpose(0, 2, 1, 3).reshape(
        B, _NT, K2)

    loss2d = pl.pallas_call(
        functools.partial(_tc_body, N, B, K2),
        out_shape=jax.ShapeDtypeStruct((1, 1), jnp.float32),
    )(cf, ci, gsb[..., None], geb[..., None], gcb[..., None])
    return loss2d[0, 0]
